# Initial kernel scaffold; baseline (speedup 1.0000x reference)
import jax
import jax.numpy as jnp
from jax.experimental import pallas as pl


def kernel(p, x, o, W1a, g1a, b1a, W1b, g1b, b1b, W2a, g2a, b2a, W2b, g2b, b2b, W3a, g3a, b3a, Wg, gg, bg):
    raise NotImplementedError("write your pallas kernel here")


# speedup vs baseline: 8.5821x; 8.5821x over previous
import functools

import jax
import jax.numpy as jnp
from jax import lax
from jax.experimental import pallas as pl
from jax.experimental.pallas import tpu as pltpu
from jax.experimental.pallas import tpu_sc as plsc

N = 16384
B = 8
NPTS = 2048
K = 16
EPS = 1e-5

F32 = jnp.float32
I32 = jnp.int32


def _leaky(h):
    return jnp.where(h >= 0, h, 0.2 * h)


_KNN_R = 256


def _knn_body(xr_ref, xt_ref, sqr_ref, sqc_ref, idx_ref):
    b = pl.program_id(0)
    xr = xr_ref[0]
    xt = xt_ref[0]
    dot = jnp.dot(xr, xt)
    d = (sqr_ref[0] + sqc_ref[0]) - 2.0 * dot

    col = lax.broadcasted_iota(I32, (_KNN_R, NPTS), 1)
    cols = []
    for _ in range(K):
        m = jnp.min(d, axis=1, keepdims=True)
        cand = jnp.where(d == m, col, NPTS)
        j = jnp.min(cand, axis=1)
        d = jnp.where(col == j[:, None], jnp.inf, d)
        cols.append(j[:, None] + b * NPTS)
    idx_ref[0] = jnp.concatenate(cols, axis=1)


def _knn(x3d, xt3, sq3r, sq3c, c):
    grid = (B, NPTS // _KNN_R)
    return pl.pallas_call(
        _knn_body,
        grid=grid,
        in_specs=[
            pl.BlockSpec((1, _KNN_R, c), lambda b, r: (b, r, 0)),
            pl.BlockSpec((1, c, NPTS), lambda b, r: (b, 0, 0)),
            pl.BlockSpec((1, _KNN_R, 1), lambda b, r: (b, r, 0)),
            pl.BlockSpec((1, 1, NPTS), lambda b, r: (b, 0, 0)),
        ],
        out_specs=pl.BlockSpec((1, _KNN_R, K), lambda b, r: (b, r, 0)),
        out_shape=jax.ShapeDtypeStruct((B, NPTS, K), I32),
    )(x3d, xt3, sq3r, sq3c)


_NC, _NS = 2, 16
_NW = _NC * _NS
_CHUNK = 128
_EDGES = N * K
_ROWS_PER_W = _EDGES // _NW
_ITERS = _ROWS_PER_W // _CHUNK


def _gather_body(idx_hbm, tab_hbm, out_hbm, idx_v, rows_v, sem):
    wid = lax.axis_index("s") * _NC + lax.axis_index("c")
    base = wid * _ROWS_PER_W

    def step(i, _):
        off = base + i * _CHUNK
        pltpu.sync_copy(idx_hbm.at[pl.ds(off, _CHUNK)], idx_v)
        pltpu.async_copy(tab_hbm.at[idx_v], rows_v, sem).wait()
        pltpu.sync_copy(rows_v, out_hbm.at[pl.ds(off, _CHUNK)])
        return _

    lax.fori_loop(0, _ITERS, step, None)


def _gather(idx_flat, table):
    mesh = plsc.VectorSubcoreMesh(core_axis_name="c", subcore_axis_name="s")
    fn = functools.partial(
        pl.kernel,
        mesh=mesh,
        out_type=jax.ShapeDtypeStruct((_EDGES, 128), F32),
        scratch_types=[
            pltpu.VMEM((_CHUNK,), I32),
            pltpu.VMEM((_CHUNK, 128), F32),
            pltpu.SemaphoreType.DMA,
        ],
    )(_gather_body)
    return fn(idx_flat, table)


_RP = 256


def _finmax_body(h_ref, mv_ref, gb_ref, x_ref):
    m, v = mv_ref[0], mv_ref[1]
    g, b = gb_ref[0], gb_ref[1]
    hn = _leaky(g * (h_ref[...] - m) / jnp.sqrt(v + EPS) + b)
    x_ref[...] = jnp.max(hn, axis=1)


def _finmax(h, mv, gb):
    grid = (N // _RP,)
    return pl.pallas_call(
        _finmax_body,
        grid=grid,
        in_specs=[
            pl.BlockSpec((_RP, K, 64), lambda i: (i, 0, 0)),
            pl.BlockSpec((2, 64), lambda i: (0, 0)),
            pl.BlockSpec((2, 64), lambda i: (0, 0)),
        ],
        out_specs=pl.BlockSpec((_RP, 64), lambda i: (i, 0)),
        out_shape=jax.ShapeDtypeStruct((N, 64), F32),
    )(h, mv, gb)


_RG = 256


def _gstat_body(x1_ref, x2_ref, x3_ref, w1_ref, w2_ref, w3_ref,
                mx_ref, st_ref):
    r = (jnp.dot(x1_ref[...], w1_ref[...])
         + jnp.dot(x2_ref[...], w2_ref[...])
         + jnp.dot(x3_ref[...], w3_ref[...]))
    m = jnp.max(r, axis=0, keepdims=True)
    s = jnp.sum(r, axis=0)
    q = jnp.sum(r * r, axis=0)
    sq = jnp.concatenate([s[None, :], q[None, :]], axis=0)

    @pl.when(pl.program_id(1) == 0)
    def _():
        mx_ref[...] = jnp.full_like(mx_ref, -jnp.inf)

    mx_ref[...] = jnp.maximum(mx_ref[...], m[None])

    @pl.when((pl.program_id(0) == 0) & (pl.program_id(1) == 0))
    def _():
        st_ref[...] = jnp.zeros_like(st_ref)

    st_ref[...] += sq


def _gstat(x1, x2, x3, w1, w2, w3):
    grid = (B, NPTS // _RG)
    return pl.pallas_call(
        _gstat_body,
        grid=grid,
        in_specs=[
            pl.BlockSpec((_RG, 64), lambda b, r: (b * (NPTS // _RG) + r, 0)),
            pl.BlockSpec((_RG, 64), lambda b, r: (b * (NPTS // _RG) + r, 0)),
            pl.BlockSpec((_RG, 64), lambda b, r: (b * (NPTS // _RG) + r, 0)),
            pl.BlockSpec((64, 1024), lambda b, r: (0, 0)),
            pl.BlockSpec((64, 1024), lambda b, r: (0, 0)),
            pl.BlockSpec((64, 1024), lambda b, r: (0, 0)),
        ],
        out_specs=[
            pl.BlockSpec((1, 1, 1024), lambda b, r: (b, 0, 0)),
            pl.BlockSpec((2, 1024), lambda b, r: (0, 0)),
        ],
        out_shape=[
            jax.ShapeDtypeStruct((B, 1, 1024), F32),
            jax.ShapeDtypeStruct((2, 1024), F32),
        ],
    )(x1, x2, x3, w1, w2, w3)


def _gfin_body(mx_ref, st_ref, g_ref, b_ref, out_ref):
    cnt = float(N)
    m = st_ref[0] / cnt
    v = st_ref[1] / cnt - m * m
    row = _leaky(g_ref[0] * (mx_ref[0] - m) / jnp.sqrt(v + EPS) + b_ref[0])
    out_ref[...] = jnp.broadcast_to(row, (NPTS, 1024))


def _gfin(mx, st, g, b):
    grid = (B,)
    return pl.pallas_call(
        _gfin_body,
        grid=grid,
        in_specs=[
            pl.BlockSpec((1, 1, 1024), lambda b: (b, 0, 0)),
            pl.BlockSpec((2, 1024), lambda b: (0, 0)),
            pl.BlockSpec((1, 1024), lambda b: (0, 0)),
            pl.BlockSpec((1, 1024), lambda b: (0, 0)),
        ],
        out_specs=pl.BlockSpec((NPTS, 1024), lambda b: (b, 0)),
        out_shape=jax.ShapeDtypeStruct((N, 1024), F32),
    )(mx, st, g, b)


def _edge_conv(x, c, layers):
    x3d = x.reshape(B, NPTS, c)
    xt3 = jnp.transpose(x3d, (0, 2, 1))
    sq = jnp.sum(x3d * x3d, axis=-1)
    idx = _knn(x3d, xt3, sq[:, :, None], sq[:, None, :], c)
    table = jnp.pad(x, ((0, 0), (0, 128 - c)))
    g = _gather(idx.reshape(_EDGES), table).reshape(N, K, 128)

    exp = jnp.broadcast_to(x[:, None, :], (N, K, c))
    h = jnp.matmul(
        jnp.concatenate([g[:, :, :c] - exp, exp], axis=-1), layers[0][0])
    for li, (w, gm, bt) in enumerate(layers):
        mv = jnp.stack([jnp.mean(h, axis=(0, 1)), jnp.var(h, axis=(0, 1))])
        if li + 1 < len(layers):
            h = jnp.matmul(
                _leaky(gm * (h - mv[0]) / jnp.sqrt(mv[1] + EPS) + bt),
                layers[li + 1][0])
        else:
            return _finmax(h, mv, jnp.stack([gm, bt]))


def kernel(p, x, o, W1a, g1a, b1a, W1b, g1b, b1b, W2a, g2a, b2a,
           W2b, g2b, b2b, W3a, g3a, b3a, Wg, gg, bg):
    del p, o
    x1 = _edge_conv(x, 6, [(W1a, g1a, b1a), (W1b, g1b, b1b)])
    x2 = _edge_conv(x1, 64, [(W2a, g2a, b2a), (W2b, g2b, b2b)])
    x3 = _edge_conv(x2, 64, [(W3a, g3a, b3a)])

    mx, stg = _gstat(x1, x2, x3, Wg[:64], Wg[64:128], Wg[128:])
    globenc = _gfin(mx, stg, gg.reshape(1, -1), bg.reshape(1, -1))

    return ((x1, x2, x3), globenc)
